# baseline (speedup 1.0000x reference)
import functools

import jax
import jax.numpy as jnp
from jax import lax
from jax.experimental import pallas as pl
from jax.experimental.pallas import tpu as pltpu
from jax.experimental.pallas import tpu_sc as plsc

N = 16384
K = 128
LANES = 16
NUM_CORES = 2
NUM_SUBCORES = 16
NUM_WORKERS = NUM_CORES * NUM_SUBCORES
ROWS_PER_WORKER = N // NUM_WORKERS
CHUNK = 64
NCHUNKS = ROWS_PER_WORKER // CHUNK
NBUF = 2
CPK = K // LANES

_mesh = plsc.VectorSubcoreMesh(
    core_axis_name="c", subcore_axis_name="s",
    num_cores=NUM_CORES, num_subcores=NUM_SUBCORES,
)


@functools.partial(
    pl.kernel,
    out_type=jax.ShapeDtypeStruct((N,), jnp.float32),
    mesh=_mesh,
    scratch_types=[
        pltpu.VMEM((CHUNK, K), jnp.float32),
        pltpu.VMEM((CHUNK, K), jnp.float32),
        pltpu.VMEM((CHUNK, K), jnp.float32),
        pltpu.VMEM((CHUNK, K), jnp.float32),
        pltpu.VMEM((ROWS_PER_WORKER,), jnp.float32),
        pltpu.SemaphoreType.DMA,
        pltpu.SemaphoreType.DMA,
    ],
    compiler_params=pltpu.CompilerParams(needs_layout_passes=False),
)
def _row_dot(gu_hbm, gi_hbm, out_hbm,
             gu_v0, gi_v0, gu_v1, gi_v1, out_v, sem0, sem1):
    wid = lax.axis_index("s") * NUM_CORES + lax.axis_index("c")
    base = wid * ROWS_PER_WORKER
    bufs = ((gu_v0, gi_v0, sem0), (gu_v1, gi_v1, sem1))
    lane = lax.iota(jnp.int32, LANES)
    perms = [lane ^ k for k in (1, 2, 4, 8)]

    def start(b, g):
        guv, giv, sem = bufs[b]
        rbase = base + g * CHUNK
        pltpu.async_copy(gu_hbm.at[pl.ds(rbase, CHUNK), :], guv, sem)
        pltpu.async_copy(gi_hbm.at[pl.ds(rbase, CHUNK), :], giv, sem)

    start(0, 0)
    start(1, 1)

    def super_body(si, carry):
        for b in range(NBUF):
            guv, giv, sem = bufs[b]
            g = si * NBUF + b
            pltpu.make_async_copy(gu_hbm.at[pl.ds(0, CHUNK), :], guv, sem).wait()
            pltpu.make_async_copy(gi_hbm.at[pl.ds(0, CHUNK), :], giv, sem).wait()
            obase = g * CHUNK

            @plsc.parallel_loop(0, CHUNK, step=1, unroll=4)
            def row_body(r, guv=guv, giv=giv, obase=obase):
                acc = guv[r, pl.ds(0, LANES)] * giv[r, pl.ds(0, LANES)]
                for c in range(1, CPK):
                    acc = acc + (guv[r, pl.ds(c * LANES, LANES)]
                                 * giv[r, pl.ds(c * LANES, LANES)])
                for p in perms:
                    acc = acc + acc.at[p].get(mode="promise_in_bounds")
                idx = jnp.full((LANES,), obase + r, jnp.int32)
                plsc.store_scatter(out_v, [idx], acc)

            @pl.when(g + NBUF < NCHUNKS)
            def _issue(b=b, g=g):
                start(b, g + NBUF)
        return carry

    lax.fori_loop(0, NCHUNKS // NBUF, super_body, 0)
    pltpu.sync_copy(out_v, out_hbm.at[pl.ds(base, ROWS_PER_WORKER)])


def kernel(gu, gi):
    return _row_dot(gu, gi)


# speedup vs baseline: 1.0168x; 1.0168x over previous
import functools

import jax
import jax.numpy as jnp
from jax import lax
from jax.experimental import pallas as pl
from jax.experimental.pallas import tpu as pltpu
from jax.experimental.pallas import tpu_sc as plsc

N = 16384
K = 128
LANES = 16
NUM_CORES = 2
NUM_SUBCORES = 16
NUM_WORKERS = NUM_CORES * NUM_SUBCORES
ROWS_PER_WORKER = N // NUM_WORKERS
CHUNK = 128
NCHUNKS = ROWS_PER_WORKER // CHUNK
NBUF = 2
CPK = K // LANES

_mesh = plsc.VectorSubcoreMesh(
    core_axis_name="c", subcore_axis_name="s",
    num_cores=NUM_CORES, num_subcores=NUM_SUBCORES,
)


@functools.partial(
    pl.kernel,
    out_type=jax.ShapeDtypeStruct((N,), jnp.float32),
    mesh=_mesh,
    scratch_types=[
        pltpu.VMEM((CHUNK, K), jnp.float32),
        pltpu.VMEM((CHUNK, K), jnp.float32),
        pltpu.VMEM((CHUNK, K), jnp.float32),
        pltpu.VMEM((CHUNK, K), jnp.float32),
        pltpu.VMEM((ROWS_PER_WORKER,), jnp.float32),
        pltpu.SemaphoreType.DMA,
        pltpu.SemaphoreType.DMA,
    ],
    compiler_params=pltpu.CompilerParams(needs_layout_passes=False),
)
def _row_dot(gu_hbm, gi_hbm, out_hbm,
             gu_v0, gi_v0, gu_v1, gi_v1, out_v, sem0, sem1):
    wid = lax.axis_index("s") * NUM_CORES + lax.axis_index("c")
    base = wid * ROWS_PER_WORKER
    bufs = ((gu_v0, gi_v0, sem0), (gu_v1, gi_v1, sem1))
    lane = lax.iota(jnp.int32, LANES)
    perms = [lane ^ k for k in (1, 2, 4, 8)]

    def start(b, g):
        guv, giv, sem = bufs[b]
        rbase = base + g * CHUNK
        pltpu.async_copy(gu_hbm.at[pl.ds(rbase, CHUNK), :], guv, sem)
        pltpu.async_copy(gi_hbm.at[pl.ds(rbase, CHUNK), :], giv, sem)

    start(0, 0)
    start(1, 1)

    def super_body(si, carry):
        for b in range(NBUF):
            guv, giv, sem = bufs[b]
            g = si * NBUF + b
            pltpu.make_async_copy(gu_hbm.at[pl.ds(0, CHUNK), :], guv, sem).wait()
            pltpu.make_async_copy(gi_hbm.at[pl.ds(0, CHUNK), :], giv, sem).wait()
            obase = g * CHUNK

            @plsc.parallel_loop(0, CHUNK, step=1, unroll=4)
            def row_body(r, guv=guv, giv=giv, obase=obase):
                acc = guv[r, pl.ds(0, LANES)] * giv[r, pl.ds(0, LANES)]
                for c in range(1, CPK):
                    acc = acc + (guv[r, pl.ds(c * LANES, LANES)]
                                 * giv[r, pl.ds(c * LANES, LANES)])
                for p in perms:
                    acc = acc + acc.at[p].get(mode="promise_in_bounds")
                idx = jnp.full((LANES,), obase + r, jnp.int32)
                plsc.store_scatter(out_v, [idx], acc)

            @pl.when(g + NBUF < NCHUNKS)
            def _issue(b=b, g=g):
                start(b, g + NBUF)
        return carry

    lax.fori_loop(0, NCHUNKS // NBUF, super_body, 0)
    pltpu.sync_copy(out_v, out_hbm.at[pl.ds(base, ROWS_PER_WORKER)])


def kernel(gu, gi):
    return _row_dot(gu, gi)
